# Initial kernel scaffold; baseline (speedup 1.0000x reference)
import jax
import jax.numpy as jnp
from jax.experimental import pallas as pl


def kernel(x, table, W1, b1, W2, b2):
    raise NotImplementedError("write your pallas kernel here")


# speedup vs baseline: 11.8185x; 11.8185x over previous
import functools

import jax
import jax.numpy as jnp
from jax import lax
from jax.experimental import pallas as pl
from jax.experimental.pallas import tpu as pltpu
from jax.experimental.pallas import tpu_sc as plsc

_B, _L, _D, _H, _C = 16384, 200, 32, 64, 2
_NC, _NS = 2, 16
_NW = _NC * _NS
_BPW = _B // _NW
_CH = 100
_INV_L = 1.0 / _L

_mesh = plsc.VectorSubcoreMesh(core_axis_name="c", subcore_axis_name="s")


@functools.partial(
    pl.kernel,
    mesh=_mesh,
    out_type=jax.ShapeDtypeStruct((_B, _D), jnp.float32),
    scratch_types=[
        pltpu.VMEM((2, 2, _CH), jnp.int32),
        pltpu.VMEM((2, _L, _D), jnp.float32),
        pltpu.VMEM((_D,), jnp.float32),
        pltpu.SemaphoreType.DMA,
    ],
    compiler_params=pltpu.CompilerParams(use_tc_tiling_on_sc=False),
)
def _pool(x_hbm, table_hbm, h_hbm, idx_v, rows_v, h_v, sem):
    c = lax.axis_index("c")
    s = lax.axis_index("s")
    wid = s * _NC + c
    base = wid * _BPW

    def fire(row, buf):
        pltpu.sync_copy(x_hbm.at[base + row], idx_v.at[buf])
        pltpu.async_copy(
            table_hbm.at[idx_v.at[buf, 0]], rows_v.at[buf, pl.ds(0, _CH)], sem
        )
        pltpu.async_copy(
            table_hbm.at[idx_v.at[buf, 1]], rows_v.at[buf, pl.ds(_CH, _CH)], sem
        )

    fire(0, 0)

    def step(g, carry):
        buf = lax.rem(g, 2)

        @pl.when(g < _BPW - 1)
        def _():
            fire(g + 1, lax.rem(g + 1, 2))

        pltpu.make_async_copy(
            table_hbm.at[pl.ds(0, _L)], rows_v.at[buf], sem
        ).wait()

        def red(jj, accs):
            accs = list(accs)
            for u in range(8):
                j = jj * 8 + u
                p = u % 4
                accs[2 * p] = accs[2 * p] + rows_v[buf, j, pl.ds(0, 16)]
                accs[2 * p + 1] = accs[2 * p + 1] + rows_v[buf, j, pl.ds(16, 16)]
            return tuple(accs)

        zero = jnp.zeros((16,), jnp.float32)
        accs = lax.fori_loop(0, _L // 8, red, (zero,) * 8)
        lo = (accs[0] + accs[2]) + (accs[4] + accs[6])
        hi = (accs[1] + accs[3]) + (accs[5] + accs[7])
        h_v[pl.ds(0, 16)] = lo * _INV_L
        h_v[pl.ds(16, 16)] = hi * _INV_L
        pltpu.sync_copy(h_v, h_hbm.at[base + g])
        return carry

    lax.fori_loop(0, _BPW, step, 0)


def _mlp_body(h_ref, w1t_ref, b1_ref, w2t_ref, b2_ref, o_ref):
    h = h_ref[...]
    z = jnp.dot(h, w1t_ref[...], preferred_element_type=jnp.float32)
    z = jnp.maximum(z + b1_ref[...], 0.0)
    o_ref[...] = jnp.dot(z, w2t_ref[...], preferred_element_type=jnp.float32) + b2_ref[...]


def _mlp(h, W1, b1, W2, b2):
    blk = 2048
    return pl.pallas_call(
        _mlp_body,
        grid=(_B // blk,),
        in_specs=[
            pl.BlockSpec((blk, _D), lambda i: (i, 0)),
            pl.BlockSpec((_D, _H), lambda i: (0, 0)),
            pl.BlockSpec((1, _H), lambda i: (0, 0)),
            pl.BlockSpec((_H, _C), lambda i: (0, 0)),
            pl.BlockSpec((1, _C), lambda i: (0, 0)),
        ],
        out_specs=pl.BlockSpec((blk, _C), lambda i: (i, 0)),
        out_shape=jax.ShapeDtypeStruct((_B, _C), jnp.float32),
    )(h, W1.T, b1.reshape(1, _H), W2.T, b2.reshape(1, _C))


def kernel(x, table, W1, b1, W2, b2):
    x3 = x.reshape(_B, 2, _CH).astype(jnp.int32)
    h = _pool(x3, table)
    return _mlp(h, W1, b1, W2, b2)


# speedup vs baseline: 15.3631x; 1.2999x over previous
import functools

import jax
import jax.numpy as jnp
from jax import lax
from jax.experimental import pallas as pl
from jax.experimental.pallas import tpu as pltpu
from jax.experimental.pallas import tpu_sc as plsc

_B, _L, _D, _H, _C = 16384, 200, 32, 64, 2
_NC, _NS = 2, 16
_NW = _NC * _NS
_BPW = _B // _NW
_CHUNKS = ((0, 128), (128, 72))
_G = 4
_NG = _BPW // _G
_GL = _G * _L
_INV_L = 1.0 / _L

_mesh = plsc.VectorSubcoreMesh(core_axis_name="c", subcore_axis_name="s")


@functools.partial(
    pl.kernel,
    mesh=_mesh,
    out_type=jax.ShapeDtypeStruct((_B, _D), jnp.float32),
    scratch_types=[
        pltpu.VMEM((2, _G, _L), jnp.int32),
        pltpu.VMEM((2, _GL, _D), jnp.float32),
        pltpu.VMEM((_G, _D), jnp.float32),
        pltpu.SemaphoreType.DMA,
    ],
    compiler_params=pltpu.CompilerParams(use_tc_tiling_on_sc=False),
)
def _pool(x_hbm, table_hbm, h_hbm, idx_v, rows_v, h_v, sem):
    c = lax.axis_index("c")
    s = lax.axis_index("s")
    wid = s * _NC + c
    base = wid * _BPW

    def fire(g, buf):
        pltpu.sync_copy(x_hbm.at[pl.ds(base + g * _G, _G)], idx_v.at[buf])
        for r in range(_G):
            for off, sz in _CHUNKS:
                pltpu.async_copy(
                    table_hbm.at[idx_v.at[buf, r, pl.ds(off, sz)]],
                    rows_v.at[buf, pl.ds(r * _L + off, sz)],
                    sem,
                )

    fire(0, 0)

    def step(g, carry):
        buf = lax.rem(g, 2)

        @pl.when(g < _NG - 1)
        def _():
            fire(g + 1, lax.rem(g + 1, 2))

        pltpu.make_async_copy(
            table_hbm.at[pl.ds(0, _GL)], rows_v.at[buf], sem
        ).wait()

        for r in range(_G):
            def red(jj, accs, r=r):
                accs = list(accs)
                for u in range(8):
                    j = r * _L + jj * 8 + u
                    p = u % 4
                    accs[2 * p] = accs[2 * p] + rows_v[buf, j, pl.ds(0, 16)]
                    accs[2 * p + 1] = accs[2 * p + 1] + rows_v[buf, j, pl.ds(16, 16)]
                return tuple(accs)

            zero = jnp.zeros((16,), jnp.float32)
            accs = lax.fori_loop(0, _L // 8, red, (zero,) * 8)
            lo = (accs[0] + accs[2]) + (accs[4] + accs[6])
            hi = (accs[1] + accs[3]) + (accs[5] + accs[7])
            h_v[r, pl.ds(0, 16)] = lo * _INV_L
            h_v[r, pl.ds(16, 16)] = hi * _INV_L

        pltpu.sync_copy(h_v, h_hbm.at[pl.ds(base + g * _G, _G)])
        return carry

    lax.fori_loop(0, _NG, step, 0)


def _mlp_body(h_ref, w1t_ref, b1_ref, w2t_ref, b2_ref, o_ref):
    h = h_ref[...]
    z = jnp.dot(h, w1t_ref[...], preferred_element_type=jnp.float32)
    z = jnp.maximum(z + b1_ref[...], 0.0)
    o_ref[...] = jnp.dot(z, w2t_ref[...], preferred_element_type=jnp.float32) + b2_ref[...]


def _mlp(h, W1, b1, W2, b2):
    blk = 2048
    return pl.pallas_call(
        _mlp_body,
        grid=(_B // blk,),
        in_specs=[
            pl.BlockSpec((blk, _D), lambda i: (i, 0)),
            pl.BlockSpec((_D, _H), lambda i: (0, 0)),
            pl.BlockSpec((1, _H), lambda i: (0, 0)),
            pl.BlockSpec((_H, _C), lambda i: (0, 0)),
            pl.BlockSpec((1, _C), lambda i: (0, 0)),
        ],
        out_specs=pl.BlockSpec((blk, _C), lambda i: (i, 0)),
        out_shape=jax.ShapeDtypeStruct((_B, _C), jnp.float32),
    )(h, W1.T, b1.reshape(1, _H), W2.T, b2.reshape(1, _C))


def kernel(x, table, W1, b1, W2, b2):
    h = _pool(x, table)
    return _mlp(h, W1, b1, W2, b2)


# speedup vs baseline: 15.3734x; 1.0007x over previous
import functools

import jax
import jax.numpy as jnp
from jax import lax
from jax.experimental import pallas as pl
from jax.experimental.pallas import tpu as pltpu
from jax.experimental.pallas import tpu_sc as plsc

_B, _L, _D, _H, _C = 16384, 200, 32, 64, 2
_NC, _NS = 2, 16
_NW = _NC * _NS
_BPW = _B // _NW
_CHUNKS = ((0, 128), (128, 72))
_G = 4
_NG = _BPW // _G
_GL = _G * _L
_INV_L = 1.0 / _L

_mesh = plsc.VectorSubcoreMesh(core_axis_name="c", subcore_axis_name="s")


@functools.partial(
    pl.kernel,
    mesh=_mesh,
    out_type=jax.ShapeDtypeStruct((_B, _D), jnp.float32),
    scratch_types=[
        pltpu.VMEM((2, _GL), jnp.int32),
        pltpu.VMEM((2, _GL, _D), jnp.float32),
        pltpu.VMEM((_G, _D), jnp.float32),
        pltpu.SemaphoreType.DMA,
    ],
    compiler_params=pltpu.CompilerParams(use_tc_tiling_on_sc=False),
)
def _pool(x_hbm, table_hbm, h_hbm, idx_v, rows_v, h_v, sem):
    c = lax.axis_index("c")
    s = lax.axis_index("s")
    wid = s * _NC + c
    base = wid * _BPW

    def fire(g, buf):
        pltpu.sync_copy(
            x_hbm.at[pl.ds((base + g * _G) * _L, _GL)], idx_v.at[buf]
        )
        for r in range(_G):
            for off, sz in _CHUNKS:
                pltpu.async_copy(
                    table_hbm.at[idx_v.at[buf, pl.ds(r * _L + off, sz)]],
                    rows_v.at[buf, pl.ds(r * _L + off, sz)],
                    sem,
                )

    fire(0, 0)

    def step(g, carry):
        buf = lax.rem(g, 2)

        @pl.when(g < _NG - 1)
        def _():
            fire(g + 1, lax.rem(g + 1, 2))

        pltpu.make_async_copy(
            table_hbm.at[pl.ds(0, _GL)], rows_v.at[buf], sem
        ).wait()

        for r in range(_G):
            def red(jj, accs, r=r):
                accs = list(accs)
                for u in range(8):
                    j = r * _L + jj * 8 + u
                    p = u % 4
                    accs[2 * p] = accs[2 * p] + rows_v[buf, j, pl.ds(0, 16)]
                    accs[2 * p + 1] = accs[2 * p + 1] + rows_v[buf, j, pl.ds(16, 16)]
                return tuple(accs)

            zero = jnp.zeros((16,), jnp.float32)
            accs = lax.fori_loop(0, _L // 8, red, (zero,) * 8)
            lo = (accs[0] + accs[2]) + (accs[4] + accs[6])
            hi = (accs[1] + accs[3]) + (accs[5] + accs[7])
            h_v[r, pl.ds(0, 16)] = lo * _INV_L
            h_v[r, pl.ds(16, 16)] = hi * _INV_L

        pltpu.sync_copy(h_v, h_hbm.at[pl.ds(base + g * _G, _G)])
        return carry

    lax.fori_loop(0, _NG, step, 0)


def _mlp_body(h_ref, w1t_ref, b1_ref, w2t_ref, b2_ref, o_ref):
    h = h_ref[...]
    z = jnp.dot(h, w1t_ref[...], preferred_element_type=jnp.float32)
    z = jnp.maximum(z + b1_ref[...], 0.0)
    o_ref[...] = jnp.dot(z, w2t_ref[...], preferred_element_type=jnp.float32) + b2_ref[...]


def _mlp(h, W1, b1, W2, b2):
    blk = 2048
    return pl.pallas_call(
        _mlp_body,
        grid=(_B // blk,),
        in_specs=[
            pl.BlockSpec((blk, _D), lambda i: (i, 0)),
            pl.BlockSpec((_D, _H), lambda i: (0, 0)),
            pl.BlockSpec((1, _H), lambda i: (0, 0)),
            pl.BlockSpec((_H, _C), lambda i: (0, 0)),
            pl.BlockSpec((1, _C), lambda i: (0, 0)),
        ],
        out_specs=pl.BlockSpec((blk, _C), lambda i: (i, 0)),
        out_shape=jax.ShapeDtypeStruct((_B, _C), jnp.float32),
    )(h, W1.T, b1.reshape(1, _H), W2.T, b2.reshape(1, _C))


def kernel(x, table, W1, b1, W2, b2):
    h = _pool(x.reshape(-1), table)
    return _mlp(h, W1, b1, W2, b2)
